# baseline (speedup 1.0000x reference)
import functools

import jax
import jax.numpy as jnp
from jax import lax
from jax.experimental import pallas as pl
from jax.experimental.pallas import tpu as pltpu
from jax.experimental.pallas import tpu_sc as plsc

B = 16384
NUM_ROWS = 1000000
L = 200
D = 64
EPS = 1e-5
LANES = 16
NC, NS = 2, 16
NW = NC * NS
RPW = B // NW
CH0, CH1 = 104, 96
WC = D // 2
NCH = 2
BR = 16
CPB = BR * NCH
NB = RPW // BR
NBUF = 16
ND = D // LANES
KACC = 4


def _sc_pool_body(idx_hbm, table_hbm, out_hbm, idx_v, rows_v, out_v,
                  isem, *gsems):
    wid = lax.axis_index("s") * NC + lax.axis_index("c")
    base = wid * RPW

    def chunk_dst(s, c):
        if c % NCH == 0:
            return rows_v.at[s]
        return rows_v.at[s, pl.ds(0, CH1)]

    def issue_gather(s, c, idx_ref):
        pltpu.async_copy(table_hbm.at[idx_ref], chunk_dst(s, c), gsems[s])

    def wait_gather(s, c):
        n = CH0 if c % NCH == 0 else CH1
        pltpu.make_async_copy(
            table_hbm.at[pl.ds(0, n)], chunk_dst(s, c), gsems[s]).wait()

    def wait_idx():
        pltpu.make_async_copy(
            idx_hbm.at[pl.ds(0, BR)], idx_v.at[0], isem).wait()

    def chunk_idx(p, c):
        r, j = divmod(c, NCH)
        if j == 0:
            return idx_v.at[p, r, pl.ds(0, CH0)]
        return idx_v.at[p, r, pl.ds(CH0, CH1)]

    def reduce_chunk(s, c, acc):
        n = CH0 if c % NCH == 0 else CH1
        def rb(l4, a):
            res = list(a)
            for u in range(KACC):
                l = l4 * KACC + u
                for q in range(WC // LANES):
                    pw = lax.bitcast_convert_type(
                        rows_v[s, l, pl.ds(q * LANES, LANES)], jnp.int32)
                    lo = lax.bitcast_convert_type(pw << 16, jnp.float32)
                    hi = lax.bitcast_convert_type(
                        pw & jnp.int32(-65536), jnp.float32)
                    k0 = u * ND + q * 2
                    res[k0] = res[k0] + lo
                    res[k0 + 1] = res[k0 + 1] + hi
            return tuple(res)
        return lax.fori_loop(0, n // KACC, rb, acc)

    def run_block(k, p, pn, tail):
        acc = None
        for c in range(CPB):
            r, j = divmod(c, NCH)
            s = c % NBUF
            if (not tail) and c == CPB - NBUF:
                wait_idx()
            wait_gather(s, c)
            if j == 0:
                acc = tuple(jnp.zeros((LANES,), jnp.float32)
                            for _ in range(ND * KACC))
            acc = reduce_chunk(s, c, acc)
            cn = c + NBUF
            if cn < CPB:
                issue_gather(s, cn, chunk_idx(p, cn))
            elif not tail:
                issue_gather(s, cn - CPB, chunk_idx(pn, cn - CPB))
            if j == NCH - 1:
                row = k * BR + r
                for d in range(ND):
                    tot = acc[d]
                    for u in range(1, KACC):
                        tot = tot + acc[u * ND + d]
                    out_v[row, pl.ds(d * LANES, LANES)] = tot

    pltpu.sync_copy(idx_hbm.at[pl.ds(base, BR)], idx_v.at[0])
    for s in range(NBUF):
        issue_gather(s, s, chunk_idx(0, s))

    def main_block(k, carry):
        p = lax.rem(k, 2)
        pn = 1 - p
        pltpu.async_copy(
            idx_hbm.at[pl.ds(base + (k + 1) * BR, BR)], idx_v.at[pn], isem)
        run_block(k, p, pn, tail=False)
        return carry

    lax.fori_loop(0, NB - 1, main_block, 0)
    run_block(NB - 1, (NB - 1) % 2, None, tail=True)
    pltpu.sync_copy(out_v, out_hbm.at[pl.ds(base, RPW)])


@functools.partial(
    pl.kernel,
    out_type=jax.ShapeDtypeStruct((B, D), jnp.float32),
    mesh=plsc.VectorSubcoreMesh(core_axis_name="c", subcore_axis_name="s"),
    compiler_params=pltpu.CompilerParams(use_tc_tiling_on_sc=False),
    scratch_types=[
        pltpu.VMEM((2, BR, L), jnp.int32),
        pltpu.VMEM((NBUF, CH0, WC), jnp.float32),
        pltpu.VMEM((RPW, D), jnp.float32),
    ] + [pltpu.SemaphoreType.DMA] * (1 + NBUF),
)
def _sc_pool(idx_hbm, table_hbm, out_hbm, idx_v, rows_v, out_v,
             isem, *gsems):
    _sc_pool_body(idx_hbm, table_hbm, out_hbm, idx_v, rows_v, out_v,
                  isem, *gsems)


FB = 8192
FB4 = FB // 4
SH4 = FB4.bit_length() - 1
NFB = (NUM_ROWS + FB - 1) // FB
NRP = NFB * FB


def _fmt_body(tt_ref, out_ref):
    ey = (lax.broadcasted_iota(jnp.int32, (D, D), 0)
          == lax.broadcasted_iota(jnp.int32, (D, D), 1)).astype(jnp.float32)
    xt = lax.dot_general(tt_ref[...], ey, (((0,), (0,)), ((), ())),
                         preferred_element_type=jnp.float32)
    u = lax.bitcast_convert_type(xt.astype(jnp.bfloat16).astype(jnp.float32),
                                 jnp.int32)
    pk2 = (lax.shift_right_logical(u[:, :WC], 16)
           | (u[:, WC:] & jnp.int32(-65536)))
    pk2 = lax.bitcast_convert_type(pk2, jnp.float32)
    out_ref[...] = jnp.concatenate(
        [lax.slice(pk2, (a * FB4, 0), ((a + 1) * FB4, WC)) for a in range(4)],
        axis=1)


def _format_table(table):
    fmt = pl.pallas_call(
        _fmt_body,
        grid=(NFB,),
        in_specs=[pl.BlockSpec((D, FB), lambda i: (0, i))],
        out_specs=pl.BlockSpec((FB4, 128), lambda i: (i, 0)),
        out_shape=jax.ShapeDtypeStruct((NRP // 4, 128), jnp.float32),
    )
    return fmt(table.T).reshape(NRP, WC)


PERM = [16 * (g // 2) + l + 32 * (g % 2)
        for g in range(ND) for l in range(LANES)]


def _tc_head_body(pooled_ref, w_ref, b_ref, g_ref, bt_ref, out_ref):
    pooled = pooled_ref[...] * (1.0 / L)
    dense = lax.dot_general(
        pooled, w_ref[...], (((1,), (1,)), ((), ())),
        preferred_element_type=jnp.float32,
        precision=lax.Precision.HIGHEST,
    ) + b_ref[...]
    mu = jnp.mean(dense, axis=0, keepdims=True)
    cent = dense - mu
    var = jnp.mean(cent * cent, axis=0, keepdims=True)
    normed = cent * (1.0 / jnp.sqrt(var + EPS)) * g_ref[...] + bt_ref[...]
    inv = 1.0 / jnp.sqrt(jnp.sum(normed * normed, axis=1, keepdims=True))
    out_ref[...] = normed * inv


def kernel(inputs, table, W, b, gamma, beta):
    tpacked = _format_table(table)
    idx = inputs.astype(jnp.int32)
    gidx = (idx & ~(FB - 1)) | ((idx & (FB4 - 1)) << 2) | ((idx >> SH4) & 3)
    pooled = _sc_pool(gidx, tpacked)
    return pl.pallas_call(
        _tc_head_body,
        out_shape=jax.ShapeDtypeStruct((B, D), jnp.float32),
    )(pooled, W[:, jnp.array(PERM)], b.reshape(1, D),
      gamma.reshape(1, D), beta.reshape(1, D))


# speedup vs baseline: 1.1335x; 1.1335x over previous
import functools

import jax
import jax.numpy as jnp
from jax import lax
from jax.experimental import pallas as pl
from jax.experimental.pallas import tpu as pltpu
from jax.experimental.pallas import tpu_sc as plsc

B = 16384
NUM_ROWS = 1000000
L = 200
D = 64
EPS = 1e-5
LANES = 16
NC, NS = 2, 16
NW = NC * NS
RPW = B // NW
CH0, CH1 = 104, 96
WC = D // 2
NCH = 2
BR = 8
CPB = BR * NCH
NB = RPW // BR
NBUF = 8
ND = D // LANES
KACC = 4


def _sc_pool_body(idx_hbm, table_hbm, out_hbm, idx_v, rows_v, out_v,
                  isem, *gsems):
    wid = lax.axis_index("s") * NC + lax.axis_index("c")
    base = wid * RPW

    def chunk_dst(s, c):
        if c % NCH == 0:
            return rows_v.at[s]
        return rows_v.at[s, pl.ds(0, CH1)]

    def issue_gather(s, c, idx_ref):
        pltpu.async_copy(table_hbm.at[idx_ref], chunk_dst(s, c), gsems[s])

    def wait_gather(s, c):
        n = CH0 if c % NCH == 0 else CH1
        pltpu.make_async_copy(
            table_hbm.at[pl.ds(0, n)], chunk_dst(s, c), gsems[s]).wait()

    def wait_idx():
        pltpu.make_async_copy(
            idx_hbm.at[pl.ds(0, BR)], idx_v.at[0], isem).wait()

    def chunk_idx(p, c):
        r, j = divmod(c, NCH)
        if j == 0:
            return idx_v.at[p, r, pl.ds(0, CH0)]
        return idx_v.at[p, r, pl.ds(CH0, CH1)]

    def reduce_chunk(s, c, acc):
        n = CH0 if c % NCH == 0 else CH1
        def rb(l4, a):
            res = list(a)
            for u in range(KACC):
                l = l4 * KACC + u
                for q in range(WC // LANES):
                    pw = lax.bitcast_convert_type(
                        rows_v[s, l, pl.ds(q * LANES, LANES)], jnp.int32)
                    lo = lax.bitcast_convert_type(pw << 16, jnp.float32)
                    hi = lax.bitcast_convert_type(
                        pw & jnp.int32(-65536), jnp.float32)
                    k0 = u * ND + q * 2
                    res[k0] = res[k0] + lo
                    res[k0 + 1] = res[k0 + 1] + hi
            return tuple(res)
        return lax.fori_loop(0, n // KACC, rb, acc)

    def run_block(k, p, pn, tail):
        acc = None
        for c in range(CPB):
            r, j = divmod(c, NCH)
            s = c % NBUF
            if (not tail) and c == CPB - NBUF:
                wait_idx()
            wait_gather(s, c)
            if j == 0:
                acc = tuple(jnp.zeros((LANES,), jnp.float32)
                            for _ in range(ND * KACC))
            acc = reduce_chunk(s, c, acc)
            cn = c + NBUF
            if cn < CPB:
                issue_gather(s, cn, chunk_idx(p, cn))
            elif not tail:
                issue_gather(s, cn - CPB, chunk_idx(pn, cn - CPB))
            if j == NCH - 1:
                row = k * BR + r
                for d in range(ND):
                    tot = acc[d]
                    for u in range(1, KACC):
                        tot = tot + acc[u * ND + d]
                    out_v[row, pl.ds(d * LANES, LANES)] = tot

    pltpu.sync_copy(idx_hbm.at[pl.ds(base, BR)], idx_v.at[0])
    for s in range(NBUF):
        issue_gather(s, s, chunk_idx(0, s))

    def main_block(k, carry):
        p = lax.rem(k, 2)
        pn = 1 - p
        pltpu.async_copy(
            idx_hbm.at[pl.ds(base + (k + 1) * BR, BR)], idx_v.at[pn], isem)
        run_block(k, p, pn, tail=False)
        return carry

    lax.fori_loop(0, NB - 1, main_block, 0)
    run_block(NB - 1, (NB - 1) % 2, None, tail=True)
    pltpu.sync_copy(out_v, out_hbm.at[pl.ds(base, RPW)])


@functools.partial(
    pl.kernel,
    out_type=jax.ShapeDtypeStruct((B, D), jnp.float32),
    mesh=plsc.VectorSubcoreMesh(core_axis_name="c", subcore_axis_name="s"),
    compiler_params=pltpu.CompilerParams(use_tc_tiling_on_sc=False),
    scratch_types=[
        pltpu.VMEM((2, BR, L), jnp.int32),
        pltpu.VMEM((NBUF, CH0, WC), jnp.float32),
        pltpu.VMEM((RPW, D), jnp.float32),
    ] + [pltpu.SemaphoreType.DMA] * (1 + NBUF),
)
def _sc_pool(idx_hbm, table_hbm, out_hbm, idx_v, rows_v, out_v,
             isem, *gsems):
    _sc_pool_body(idx_hbm, table_hbm, out_hbm, idx_v, rows_v, out_v,
                  isem, *gsems)


FB = 8192
FB4 = FB // 4
SH4 = FB4.bit_length() - 1
NFB = (NUM_ROWS + FB - 1) // FB
NRP = NFB * FB


def _fmt_body(tt_ref, out_ref):
    xb = tt_ref[...].astype(jnp.bfloat16)
    pk = pltpu.bitcast(xb, jnp.float32)
    pk2 = jnp.swapaxes(pk, 0, 1)
    out_ref[...] = jnp.concatenate(
        [lax.slice(pk2, (a * FB4, 0), ((a + 1) * FB4, WC)) for a in range(4)],
        axis=1)


def _format_table(table):
    fmt = pl.pallas_call(
        _fmt_body,
        grid=(NFB,),
        in_specs=[pl.BlockSpec((D, FB), lambda i: (0, i))],
        out_specs=pl.BlockSpec((FB4, 128), lambda i: (i, 0)),
        out_shape=jax.ShapeDtypeStruct((NRP // 4, 128), jnp.float32),
    )
    return fmt(table.T).reshape(NRP, WC)


PERM = [32 * (g // 2) + 2 * l + (g % 2)
        for g in range(ND) for l in range(LANES)]


def _tc_head_body(pooled_ref, w_ref, b_ref, g_ref, bt_ref, out_ref):
    pooled = pooled_ref[...] * (1.0 / L)
    dense = lax.dot_general(
        pooled, w_ref[...], (((1,), (1,)), ((), ())),
        preferred_element_type=jnp.float32,
        precision=lax.Precision.HIGHEST,
    ) + b_ref[...]
    mu = jnp.mean(dense, axis=0, keepdims=True)
    cent = dense - mu
    var = jnp.mean(cent * cent, axis=0, keepdims=True)
    normed = cent * (1.0 / jnp.sqrt(var + EPS)) * g_ref[...] + bt_ref[...]
    inv = 1.0 / jnp.sqrt(jnp.sum(normed * normed, axis=1, keepdims=True))
    out_ref[...] = normed * inv


def kernel(inputs, table, W, b, gamma, beta):
    tpacked = _format_table(table)
    idx = inputs.astype(jnp.int32)
    gidx = (idx & ~(FB - 1)) | ((idx & (FB4 - 1)) << 2) | ((idx >> SH4) & 3)
    pooled = _sc_pool(gidx, tpacked)
    return pl.pallas_call(
        _tc_head_body,
        out_shape=jax.ShapeDtypeStruct((B, D), jnp.float32),
    )(pooled, W[:, jnp.array(PERM)], b.reshape(1, D),
      gamma.reshape(1, D), beta.reshape(1, D))
